# baseline (device time: 12897 ns/iter reference)
import jax
import jax.numpy as jnp
from jax import lax
from jax.experimental import pallas as pl
from jax.experimental.pallas import tpu as pltpu


def kernel(A, B):
    m, k = A.shape
    _, n = B.shape

    def body(a_ref, b_ref, out_ref, a_bf, b_bf, a_rcv, b_rcv,
             send_sems, recv_sems):
        my_x = lax.axis_index("x")
        my_y = lax.axis_index("y")
        partner = (1 - my_x, my_y)

        a_bf[...] = a_ref[...].astype(jnp.bfloat16)
        b_bf[...] = b_ref[...].astype(jnp.bfloat16)

        barrier_sem = pltpu.get_barrier_semaphore()
        pl.semaphore_signal(
            barrier_sem, inc=1,
            device_id=partner, device_id_type=pl.DeviceIdType.MESH,
        )
        pl.semaphore_wait(barrier_sem, 1)

        rdma_a = pltpu.make_async_remote_copy(
            src_ref=a_bf, dst_ref=a_rcv,
            send_sem=send_sems.at[0], recv_sem=recv_sems.at[0],
            device_id=partner, device_id_type=pl.DeviceIdType.MESH,
        )
        rdma_b = pltpu.make_async_remote_copy(
            src_ref=b_bf, dst_ref=b_rcv,
            send_sem=send_sems.at[1], recv_sem=recv_sems.at[1],
            device_id=partner, device_id_type=pl.DeviceIdType.MESH,
        )
        rdma_a.start()
        rdma_b.start()

        out_ref[...] = jnp.dot(
            a_bf[...], b_bf[...], preferred_element_type=jnp.float32
        )

        rdma_a.wait()
        rdma_b.wait()
        out_ref[...] += jnp.dot(
            a_rcv[...], b_rcv[...], preferred_element_type=jnp.float32
        )

    return pl.pallas_call(
        body,
        out_shape=jax.ShapeDtypeStruct((m, n), jnp.float32),
        in_specs=[
            pl.BlockSpec(memory_space=pltpu.VMEM),
            pl.BlockSpec(memory_space=pltpu.VMEM),
        ],
        out_specs=pl.BlockSpec(memory_space=pltpu.VMEM),
        scratch_shapes=[
            pltpu.VMEM((m, k), jnp.bfloat16),
            pltpu.VMEM((k, n), jnp.bfloat16),
            pltpu.VMEM((m, k), jnp.bfloat16),
            pltpu.VMEM((k, n), jnp.bfloat16),
            pltpu.SemaphoreType.DMA((2,)),
            pltpu.SemaphoreType.DMA((2,)),
        ],
        compiler_params=pltpu.CompilerParams(collective_id=0),
    )(A, B)


# device time: 12846 ns/iter; 1.0040x vs baseline; 1.0040x over previous
import jax
import jax.numpy as jnp
from jax import lax
from jax.experimental import pallas as pl
from jax.experimental.pallas import tpu as pltpu


N_CHUNKS = 4


def kernel(A, B):
    m, k = A.shape
    _, n = B.shape
    nc = n // N_CHUNKS

    def body(a_ref, b_ref, out_ref, a_bf, b_bf, a_rcv, b_rcv,
             send_sems, recv_sems):
        my_x = lax.axis_index("x")
        my_y = lax.axis_index("y")
        partner = (1 - my_x, my_y)

        a_bf[...] = a_ref[...].astype(jnp.bfloat16)
        for j in range(N_CHUNKS):
            b_bf[j] = b_ref[:, pl.ds(j * nc, nc)].astype(jnp.bfloat16)

        barrier_sem = pltpu.get_barrier_semaphore()
        pl.semaphore_signal(
            barrier_sem, inc=1,
            device_id=partner, device_id_type=pl.DeviceIdType.MESH,
        )
        pl.semaphore_wait(barrier_sem, 1)

        rdma_a = pltpu.make_async_remote_copy(
            src_ref=a_bf, dst_ref=a_rcv,
            send_sem=send_sems.at[0], recv_sem=recv_sems.at[0],
            device_id=partner, device_id_type=pl.DeviceIdType.MESH,
        )
        rdma_a.start()
        rdma_bs = []
        for j in range(N_CHUNKS):
            r = pltpu.make_async_remote_copy(
                src_ref=b_bf.at[j], dst_ref=b_rcv.at[j],
                send_sem=send_sems.at[1 + j], recv_sem=recv_sems.at[1 + j],
                device_id=partner, device_id_type=pl.DeviceIdType.MESH,
            )
            r.start()
            rdma_bs.append(r)

        for j in range(N_CHUNKS):
            out_ref[:, pl.ds(j * nc, nc)] = jnp.dot(
                a_bf[...], b_bf[j], preferred_element_type=jnp.float32
            )

        rdma_a.wait_recv()
        for j in range(N_CHUNKS):
            rdma_bs[j].wait_recv()
            out_ref[:, pl.ds(j * nc, nc)] += jnp.dot(
                a_rcv[...], b_rcv[j], preferred_element_type=jnp.float32
            )

        rdma_a.wait_send()
        for j in range(N_CHUNKS):
            rdma_bs[j].wait_send()

    return pl.pallas_call(
        body,
        out_shape=jax.ShapeDtypeStruct((m, n), jnp.float32),
        in_specs=[
            pl.BlockSpec(memory_space=pltpu.VMEM),
            pl.BlockSpec(memory_space=pltpu.VMEM),
        ],
        out_specs=pl.BlockSpec(memory_space=pltpu.VMEM),
        scratch_shapes=[
            pltpu.VMEM((m, k), jnp.bfloat16),
            pltpu.VMEM((N_CHUNKS, k, nc), jnp.bfloat16),
            pltpu.VMEM((m, k), jnp.bfloat16),
            pltpu.VMEM((N_CHUNKS, k, nc), jnp.bfloat16),
            pltpu.SemaphoreType.DMA((1 + N_CHUNKS,)),
            pltpu.SemaphoreType.DMA((1 + N_CHUNKS,)),
        ],
        compiler_params=pltpu.CompilerParams(collective_id=0),
    )(A, B)
